# baseline (speedup 1.0000x reference)
import functools

import jax
import jax.numpy as jnp
from jax import lax
from jax.experimental import pallas as pl
from jax.experimental.pallas import tpu as pltpu
from jax.experimental.pallas import tpu_sc as plsc

N = 10000
E = 160000
D = 256
H = 128
NTILES = 16
CHUNK = 64
PER_TILE = 10240
NCHUNKS = PER_TILE // CHUNK
GROUP = 40
NGROUPS = NCHUNKS // GROUP
EPAD = PER_TILE * NTILES
AROWS = 10240
ZROWS = AROWS // NTILES

_mesh = plsc.VectorSubcoreMesh(core_axis_name="c", subcore_axis_name="s")


@functools.partial(
    pl.kernel,
    mesh=_mesh,
    out_type=jax.ShapeDtypeStruct((2, AROWS, H), jnp.float32),
    scratch_types=[
        pltpu.VMEM_SHARED((AROWS, H), jnp.float32),
        pltpu.VMEM((GROUP, CHUNK), jnp.int32),
        pltpu.VMEM((GROUP, CHUNK), jnp.int32),
        pltpu.VMEM((GROUP, CHUNK), jnp.int32),
        pltpu.VMEM((CHUNK, H), jnp.float32),
        pltpu.VMEM((CHUNK, H), jnp.float32),
        pltpu.VMEM((CHUNK, H), jnp.float32),
        pltpu.VMEM((CHUNK, H), jnp.float32),
        pltpu.SemaphoreType.DMA,
        pltpu.SemaphoreType.DMA,
        pltpu.SemaphoreType.DMA,
    ],
)
def _sc_message_pass(x_il, src3, dst3, cmb3, ctab_rep, out,
                     accum, src_v, dst_v, cmb_v,
                     rows0, rows1, emb0, emb1, sem0, sem1, semz):
    c = lax.axis_index("c")
    s = lax.axis_index("s")

    zero = jnp.zeros((16,), jnp.float32)

    @plsc.parallel_loop(0, CHUNK, unroll=4)
    def _zfill(i):
        for j in range(H // 16):
            rows0[i, pl.ds(j * 16, 16)] = zero
    zcopies = [
        pltpu.async_copy(rows0, accum.at[pl.ds(s * ZROWS + k * CHUNK, CHUNK)],
                         semz)
        for k in range(ZROWS // CHUNK)
    ]
    for zc in zcopies:
        zc.wait()
    plsc.subcore_barrier()

    rows = (rows0, rows1)
    emb = (emb0, emb1)
    sem = (sem0, sem1)

    for grp in range(NGROUPS):
        gbase = grp * GROUP
        pltpu.sync_copy(src3.at[s, pl.ds(gbase, GROUP)], src_v)
        pltpu.sync_copy(dst3.at[s, pl.ds(gbase, GROUP)], dst_v)
        pltpu.sync_copy(cmb3.at[s, pl.ds(gbase, GROUP)], cmb_v)

        @plsc.parallel_loop(0, GROUP, unroll=4)
        def _xform(i):
            for j in range(CHUNK // 16):
                sl = pl.ds(j * 16, 16)
                src_v[i, sl] = src_v[i, sl] * 2 + c
                cmb_v[i, sl] = (cmb_v[i, sl] * 2 + c + s * 36 + c * 576
                                + (i & 3) * 1152)

        for b in range(2):
            pltpu.async_copy(x_il.at[src_v.at[b]], rows[b], sem[b])
            pltpu.async_copy(ctab_rep.at[cmb_v.at[b]], emb[b], sem[b])

        def _pair(p, carry):
            for b in range(2):
                gi = p * 2 + b
                pltpu.make_async_copy(
                    x_il.at[src_v.at[gi]], rows[b], sem[b]).wait()
                pltpu.make_async_copy(
                    x_il.at[src_v.at[gi]], emb[b], sem[b]).wait()

                @plsc.parallel_loop(0, CHUNK, unroll=4)
                def _mul(i):
                    for j in range(H // 16):
                        sl = pl.ds(j * 16, 16)
                        rows[b][i, sl] = rows[b][i, sl] * emb[b][i, sl]

                pltpu.sync_copy(rows[b], accum.at[dst_v.at[gi]], add=True)

                @pl.when(gi + 2 < GROUP)
                def _prefetch():
                    pltpu.async_copy(
                        x_il.at[src_v.at[gi + 2]], rows[b], sem[b])
                    pltpu.async_copy(
                        ctab_rep.at[cmb_v.at[gi + 2]], emb[b], sem[b])
            return carry
        lax.fori_loop(0, GROUP // 2, _pair, 0)

    plsc.subcore_barrier()
    pltpu.sync_copy(accum.at[pl.ds(s * ZROWS, ZROWS)],
                    out.at[c, pl.ds(s * ZROWS, ZROWS)])


def _mlp_body(parts_ref, x_ref, cself_ref, w1_ref, b1_ref, w2_ref, b2_ref,
              out_ref):
    a = jnp.concatenate([parts_ref[0], parts_ref[1]], axis=1)
    a = a + x_ref[...] * cself_ref[...]
    h = jnp.maximum(
        jnp.dot(a, w1_ref[...], preferred_element_type=jnp.float32)
        + b1_ref[...], 0.0)
    out_ref[...] = (
        jnp.dot(h, w2_ref[...], preferred_element_type=jnp.float32)
        + b2_ref[...])


_ROWS_BLK = 1000


def kernel(x, edge_index, edge_attr, emb1, emb2, W1, b1, W2, b2):
    x_il = x.reshape(N, 2, H).reshape(2 * N, H)
    ctab = (emb1[:, None, :] + emb2[None, :, :]).reshape(-1, D)
    ctab_il = jnp.tile(ctab.reshape(-1, 2, H).reshape(-1, H),
                       (8 * NTILES, 1))
    cself = (emb1[4] + emb2[0]).reshape(1, D)

    pad = EPAD - E
    cmb = edge_attr[:, 0] * 3 + edge_attr[:, 1]
    src3 = jnp.pad(edge_index[0], (0, pad)).reshape(NTILES, NCHUNKS, CHUNK)
    dst3 = jnp.pad(edge_index[1], (0, pad), constant_values=N).reshape(
        NTILES, NCHUNKS, CHUNK)
    cmb3 = jnp.pad(cmb, (0, pad)).reshape(NTILES, NCHUNKS, CHUNK)

    parts = _sc_message_pass(x_il, src3, dst3, cmb3, ctab_il)

    out = pl.pallas_call(
        _mlp_body,
        grid=(N // _ROWS_BLK,),
        in_specs=[
            pl.BlockSpec((2, _ROWS_BLK, H), lambda i: (0, i, 0)),
            pl.BlockSpec((_ROWS_BLK, D), lambda i: (i, 0)),
            pl.BlockSpec((1, D), lambda i: (0, 0)),
            pl.BlockSpec((D, 2 * D), lambda i: (0, 0)),
            pl.BlockSpec((1, 2 * D), lambda i: (0, 0)),
            pl.BlockSpec((2 * D, D), lambda i: (0, 0)),
            pl.BlockSpec((1, D), lambda i: (0, 0)),
        ],
        out_specs=pl.BlockSpec((_ROWS_BLK, D), lambda i: (i, 0)),
        out_shape=jax.ShapeDtypeStruct((N, D), jnp.float32),
    )(parts, x, cself, W1, b1.reshape(1, 2 * D), W2, b2.reshape(1, D))
    return out


# speedup vs baseline: 1.0083x; 1.0083x over previous
import functools

import jax
import jax.numpy as jnp
from jax import lax
from jax.experimental import pallas as pl
from jax.experimental.pallas import tpu as pltpu
from jax.experimental.pallas import tpu_sc as plsc

N = 10000
E = 160000
D = 256
H = 128
NTILES = 16
CHUNK = 64
PER_TILE = 10240
NCHUNKS = PER_TILE // CHUNK
GROUP = 40
NGROUPS = NCHUNKS // GROUP
EPAD = PER_TILE * NTILES
AROWS = 10240
ZROWS = AROWS // NTILES

_mesh = plsc.VectorSubcoreMesh(core_axis_name="c", subcore_axis_name="s")


@functools.partial(
    pl.kernel,
    mesh=_mesh,
    out_type=jax.ShapeDtypeStruct((2, AROWS, H), jnp.float32),
    scratch_types=[
        pltpu.VMEM_SHARED((AROWS, H), jnp.float32),
        pltpu.VMEM((GROUP, CHUNK), jnp.int32),
        pltpu.VMEM((GROUP, CHUNK), jnp.int32),
        pltpu.VMEM((GROUP, CHUNK), jnp.int32),
        pltpu.VMEM((CHUNK, H), jnp.float32),
        pltpu.VMEM((CHUNK, H), jnp.float32),
        pltpu.VMEM((CHUNK, H), jnp.float32),
        pltpu.VMEM((CHUNK, H), jnp.float32),
        pltpu.SemaphoreType.DMA,
        pltpu.SemaphoreType.DMA,
        pltpu.SemaphoreType.DMA,
    ],
)
def _sc_message_pass(x_il, src3, dst3, cmb3, ctab_rep, out,
                     accum, src_v, dst_v, cmb_v,
                     rows0, rows1, emb0, emb1, sem0, sem1, semz):
    c = lax.axis_index("c")
    s = lax.axis_index("s")

    zero = jnp.zeros((16,), jnp.float32)

    @plsc.parallel_loop(0, CHUNK, unroll=4)
    def _zfill(i):
        for j in range(H // 16):
            rows0[i, pl.ds(j * 16, 16)] = zero
    zcopies = [
        pltpu.async_copy(rows0, accum.at[pl.ds(s * ZROWS + k * CHUNK, CHUNK)],
                         semz)
        for k in range(ZROWS // CHUNK)
    ]
    for zc in zcopies:
        zc.wait()
    plsc.subcore_barrier()

    rows = (rows0, rows1)
    emb = (emb0, emb1)
    sem = (sem0, sem1)

    for grp in range(NGROUPS):
        gbase = grp * GROUP
        pltpu.sync_copy(src3.at[s, pl.ds(gbase, GROUP)], src_v)
        pltpu.sync_copy(dst3.at[s, pl.ds(gbase, GROUP)], dst_v)
        pltpu.sync_copy(cmb3.at[s, pl.ds(gbase, GROUP)], cmb_v)

        @plsc.parallel_loop(0, GROUP, unroll=4)
        def _xform(i):
            for j in range(CHUNK // 16):
                sl = pl.ds(j * 16, 16)
                src_v[i, sl] = src_v[i, sl] * 2 + c
                cmb_v[i, sl] = (cmb_v[i, sl] * 2 + c + s * 36 + c * 576
                                + (i & 1) * 1152)

        for b in range(2):
            pltpu.async_copy(x_il.at[src_v.at[b]], rows[b], sem[b])
            pltpu.async_copy(ctab_rep.at[cmb_v.at[b]], emb[b], sem[b])

        def _pair(p, carry):
            for b in range(2):
                gi = p * 2 + b
                pltpu.make_async_copy(
                    x_il.at[src_v.at[gi]], rows[b], sem[b]).wait()
                pltpu.make_async_copy(
                    x_il.at[src_v.at[gi]], emb[b], sem[b]).wait()

                @plsc.parallel_loop(0, CHUNK, unroll=4)
                def _mul(i):
                    for j in range(H // 16):
                        sl = pl.ds(j * 16, 16)
                        rows[b][i, sl] = rows[b][i, sl] * emb[b][i, sl]

                pltpu.sync_copy(rows[b], accum.at[dst_v.at[gi]], add=True)

                @pl.when(gi + 2 < GROUP)
                def _prefetch():
                    pltpu.async_copy(
                        x_il.at[src_v.at[gi + 2]], rows[b], sem[b])
                    pltpu.async_copy(
                        ctab_rep.at[cmb_v.at[gi + 2]], emb[b], sem[b])
            return carry
        lax.fori_loop(0, GROUP // 2, _pair, 0)

    plsc.subcore_barrier()
    pltpu.sync_copy(accum.at[pl.ds(s * ZROWS, ZROWS)],
                    out.at[c, pl.ds(s * ZROWS, ZROWS)])


def _mlp_body(parts_ref, x_ref, cself_ref, w1_ref, b1_ref, w2_ref, b2_ref,
              out_ref):
    a = jnp.concatenate([parts_ref[0], parts_ref[1]], axis=1)
    a = a + x_ref[...] * cself_ref[...]
    h = jnp.maximum(
        jnp.dot(a, w1_ref[...], preferred_element_type=jnp.float32)
        + b1_ref[...], 0.0)
    out_ref[...] = (
        jnp.dot(h, w2_ref[...], preferred_element_type=jnp.float32)
        + b2_ref[...])


_ROWS_BLK = 1000


def kernel(x, edge_index, edge_attr, emb1, emb2, W1, b1, W2, b2):
    x_il = x.reshape(N, 2, H).reshape(2 * N, H)
    ctab = (emb1[:, None, :] + emb2[None, :, :]).reshape(-1, D)
    ctab_il = jnp.tile(ctab.reshape(-1, 2, H).reshape(-1, H),
                       (4 * NTILES, 1))
    cself = (emb1[4] + emb2[0]).reshape(1, D)

    pad = EPAD - E
    cmb = edge_attr[:, 0] * 3 + edge_attr[:, 1]
    src3 = jnp.pad(edge_index[0], (0, pad)).reshape(NTILES, NCHUNKS, CHUNK)
    dst3 = jnp.pad(edge_index[1], (0, pad), constant_values=N).reshape(
        NTILES, NCHUNKS, CHUNK)
    cmb3 = jnp.pad(cmb, (0, pad)).reshape(NTILES, NCHUNKS, CHUNK)

    parts = _sc_message_pass(x_il, src3, dst3, cmb3, ctab_il)

    out = pl.pallas_call(
        _mlp_body,
        grid=(N // _ROWS_BLK,),
        in_specs=[
            pl.BlockSpec((2, _ROWS_BLK, H), lambda i: (0, i, 0)),
            pl.BlockSpec((_ROWS_BLK, D), lambda i: (i, 0)),
            pl.BlockSpec((1, D), lambda i: (0, 0)),
            pl.BlockSpec((D, 2 * D), lambda i: (0, 0)),
            pl.BlockSpec((1, 2 * D), lambda i: (0, 0)),
            pl.BlockSpec((2 * D, D), lambda i: (0, 0)),
            pl.BlockSpec((1, D), lambda i: (0, 0)),
        ],
        out_specs=pl.BlockSpec((_ROWS_BLK, D), lambda i: (i, 0)),
        out_shape=jax.ShapeDtypeStruct((N, D), jnp.float32),
    )(parts, x, cself, W1, b1.reshape(1, 2 * D), W2, b2.reshape(1, D))
    return out
